# baseline (device time: 47257 ns/iter reference)
import jax
import jax.numpy as jnp
from jax import lax
from jax.experimental import pallas as pl
from jax.experimental.pallas import tpu as pltpu

N_DEV = 16
SQ = 512
D_MODEL = 1024
HQ = 8
DH = 128
CHUNK = SQ // N_DEV
SCALE = 0.08838834764831843


def kernel(x, Wq, Wo, Wk, Wv):
    x2 = x.reshape(SQ, D_MODEL)

    def body(x_ref, wq_ref, wo_ref, wk_ref, wv_ref, out_ref,
             attn_ref, part_bf_ref, rs_ref, ag_ref,
             rs_send, rs_recv, ag_send, ag_recv):
        me = lax.axis_index("i")

        barrier_sem = pltpu.get_barrier_semaphore()
        for d in range(1, N_DEV):
            peer = (me + d) % N_DEV
            pl.semaphore_signal(barrier_sem, inc=1, device_id=(peer,),
                                device_id_type=pl.DeviceIdType.MESH)
        pl.semaphore_wait(barrier_sem, N_DEV - 1)

        xv = x_ref[...]
        q = jnp.dot(xv, wq_ref[...], preferred_element_type=jnp.float32)
        k = jnp.dot(xv, wk_ref[...], preferred_element_type=jnp.float32)
        v = jnp.dot(xv, wv_ref[...], preferred_element_type=jnp.float32)
        for h in range(HQ):
            sl = slice(h * DH, (h + 1) * DH)
            qh, kh, vh = q[:, sl], k[:, sl], v[:, sl]
            s = lax.dot_general(qh, kh, (((1,), (1,)), ((), ())),
                                preferred_element_type=jnp.float32) * SCALE
            m = jnp.max(s, axis=-1, keepdims=True)
            p = jnp.exp(s - m)
            denom = jnp.sum(p, axis=-1, keepdims=True)
            attn_ref[:, sl] = (
                jnp.dot(p, vh, preferred_element_type=jnp.float32) / denom
            )
        out_ref[...] = jnp.dot(attn_ref[...], wo_ref[...],
                               preferred_element_type=jnp.float32)

        my_rows = pl.ds(me * CHUNK, CHUNK)
        part_bf_ref[...] = out_ref[...].astype(jnp.bfloat16)
        rs_ref[my_rows, :] = jnp.zeros((CHUNK, D_MODEL), jnp.bfloat16)

        send_descs = []
        for d in range(1, N_DEV):
            peer = (me + d) % N_DEV
            desc = pltpu.make_async_remote_copy(
                src_ref=part_bf_ref.at[pl.ds(peer * CHUNK, CHUNK), :],
                dst_ref=rs_ref.at[pl.ds(me * CHUNK, CHUNK), :],
                send_sem=rs_send.at[peer],
                recv_sem=rs_recv.at[me],
                device_id=(peer,),
                device_id_type=pl.DeviceIdType.MESH,
            )
            desc.start()
            send_descs.append(desc)
        for d in range(1, N_DEV):
            peer = (me + d) % N_DEV
            pltpu.make_async_remote_copy(
                src_ref=part_bf_ref.at[pl.ds(peer * CHUNK, CHUNK), :],
                dst_ref=rs_ref.at[pl.ds(peer * CHUNK, CHUNK), :],
                send_sem=rs_send.at[peer],
                recv_sem=rs_recv.at[peer],
                device_id=(peer,),
                device_id_type=pl.DeviceIdType.MESH,
            ).wait_recv()

        reduced = out_ref[my_rows, :]
        for i in range(N_DEV):
            reduced = reduced + rs_ref[i * CHUNK:(i + 1) * CHUNK, :].astype(
                jnp.float32)

        ag_ref[my_rows, :] = reduced.astype(jnp.bfloat16)
        for d in range(1, N_DEV):
            peer = (me + d) % N_DEV
            desc = pltpu.make_async_remote_copy(
                src_ref=ag_ref.at[pl.ds(me * CHUNK, CHUNK), :],
                dst_ref=ag_ref.at[pl.ds(me * CHUNK, CHUNK), :],
                send_sem=ag_send.at[peer],
                recv_sem=ag_recv.at[me],
                device_id=(peer,),
                device_id_type=pl.DeviceIdType.MESH,
            )
            desc.start()
            send_descs.append(desc)
        for d in range(1, N_DEV):
            peer = (me + d) % N_DEV
            pltpu.make_async_remote_copy(
                src_ref=ag_ref.at[pl.ds(peer * CHUNK, CHUNK), :],
                dst_ref=ag_ref.at[pl.ds(peer * CHUNK, CHUNK), :],
                send_sem=ag_send.at[peer],
                recv_sem=ag_recv.at[peer],
                device_id=(peer,),
                device_id_type=pl.DeviceIdType.MESH,
            ).wait_recv()

        out_ref[...] = ag_ref[...].astype(jnp.float32)
        out_ref[my_rows, :] = reduced

        for desc in send_descs:
            desc.wait_send()

    out = pl.pallas_call(
        body,
        out_shape=jax.ShapeDtypeStruct((SQ, D_MODEL), jnp.float32),
        in_specs=[pl.BlockSpec(memory_space=pltpu.VMEM)] * 5,
        out_specs=pl.BlockSpec(memory_space=pltpu.VMEM),
        scratch_shapes=[
            pltpu.VMEM((SQ, D_MODEL), jnp.float32),
            pltpu.VMEM((SQ, D_MODEL), jnp.bfloat16),
            pltpu.VMEM((SQ, D_MODEL), jnp.bfloat16),
            pltpu.VMEM((SQ, D_MODEL), jnp.bfloat16),
            pltpu.SemaphoreType.DMA((N_DEV,)),
            pltpu.SemaphoreType.DMA((N_DEV,)),
            pltpu.SemaphoreType.DMA((N_DEV,)),
            pltpu.SemaphoreType.DMA((N_DEV,)),
        ],
        compiler_params=pltpu.CompilerParams(collective_id=0),
    )(x2, Wq, Wo, Wk, Wv)
    return out.reshape(1, SQ, D_MODEL)


# device time: 27087 ns/iter; 1.7446x vs baseline; 1.7446x over previous
import jax
import jax.numpy as jnp
from jax import lax
from jax.experimental import pallas as pl
from jax.experimental.pallas import tpu as pltpu

N_DEV = 16
SQ = 512
D_MODEL = 1024
HQ = 8
DH = 128
G = 2
ROWS_G = SQ // G
SLAB = ROWS_G // N_DEV
SCALE = 0.08838834764831843


def kernel(x, Wq, Wo, Wk, Wv):
    x2 = x.reshape(SQ, D_MODEL)

    def body(x_ref, wq_ref, wo_ref, wk_ref, wv_ref, out_ref,
             attn_ref, part_bf_ref, rs_ref, ag_ref,
             rs_send, rs_recv, ag_send, ag_recv):
        me = lax.axis_index("i")

        barrier_sem = pltpu.get_barrier_semaphore()
        for d in range(1, N_DEV):
            peer = (me + d) % N_DEV
            pl.semaphore_signal(barrier_sem, inc=1, device_id=(peer,),
                                device_id_type=pl.DeviceIdType.MESH)
        pl.semaphore_wait(barrier_sem, N_DEV - 1)

        xv = x_ref[...]
        k = jnp.dot(xv, wk_ref[...], preferred_element_type=jnp.float32)
        v = jnp.dot(xv, wv_ref[...], preferred_element_type=jnp.float32)

        send_descs = []

        def compute_group(g):
            rsl = slice(g * ROWS_G, (g + 1) * ROWS_G)
            qg = jnp.dot(xv[rsl, :], wq_ref[...],
                         preferred_element_type=jnp.float32)
            for h in range(HQ):
                csl = slice(h * DH, (h + 1) * DH)
                qh, kh, vh = qg[:, csl], k[:, csl], v[:, csl]
                s = lax.dot_general(qh, kh, (((1,), (1,)), ((), ())),
                                    preferred_element_type=jnp.float32) * SCALE
                m = jnp.max(s, axis=-1, keepdims=True)
                p = jnp.exp(s - m)
                denom = jnp.sum(p, axis=-1, keepdims=True)
                attn_ref[rsl, csl] = (
                    jnp.dot(p, vh, preferred_element_type=jnp.float32) / denom
                )
            pg = jnp.dot(attn_ref[rsl, :], wo_ref[...],
                         preferred_element_type=jnp.float32)
            part_bf_ref[rsl, :] = pg.astype(jnp.bfloat16)
            rs_ref[pl.ds(g * ROWS_G + me * SLAB, SLAB), :] = (
                part_bf_ref[pl.ds(g * ROWS_G + me * SLAB, SLAB), :])
            for d in range(1, N_DEV):
                peer = (me + d) % N_DEV
                desc = pltpu.make_async_remote_copy(
                    src_ref=part_bf_ref.at[
                        pl.ds(g * ROWS_G + peer * SLAB, SLAB), :],
                    dst_ref=rs_ref.at[pl.ds(g * ROWS_G + me * SLAB, SLAB), :],
                    send_sem=rs_send.at[g, peer],
                    recv_sem=rs_recv.at[g, me],
                    device_id=(peer,),
                    device_id_type=pl.DeviceIdType.MESH,
                )
                desc.start()
                send_descs.append(desc)

        def reduce_and_bcast(g):
            for d in range(1, N_DEV):
                peer = (me + d) % N_DEV
                pltpu.make_async_remote_copy(
                    src_ref=part_bf_ref.at[
                        pl.ds(g * ROWS_G + peer * SLAB, SLAB), :],
                    dst_ref=rs_ref.at[pl.ds(g * ROWS_G + peer * SLAB, SLAB), :],
                    send_sem=rs_send.at[g, peer],
                    recv_sem=rs_recv.at[g, peer],
                    device_id=(peer,),
                    device_id_type=pl.DeviceIdType.MESH,
                ).wait_recv()
            red = jnp.zeros((SLAB, D_MODEL), jnp.float32)
            for s in range(N_DEV):
                rows = slice(g * ROWS_G + s * SLAB, g * ROWS_G + (s + 1) * SLAB)
                red = red + rs_ref[rows, :].astype(jnp.float32)
            ag_ref[pl.ds(g * ROWS_G + me * SLAB, SLAB), :] = red.astype(
                jnp.bfloat16)
            for d in range(1, N_DEV):
                peer = (me + d) % N_DEV
                desc = pltpu.make_async_remote_copy(
                    src_ref=ag_ref.at[pl.ds(g * ROWS_G + me * SLAB, SLAB), :],
                    dst_ref=ag_ref.at[pl.ds(g * ROWS_G + me * SLAB, SLAB), :],
                    send_sem=ag_send.at[g, peer],
                    recv_sem=ag_recv.at[g, me],
                    device_id=(peer,),
                    device_id_type=pl.DeviceIdType.MESH,
                )
                desc.start()
                send_descs.append(desc)

        compute_group(0)
        compute_group(1)
        reduce_and_bcast(0)
        reduce_and_bcast(1)

        for g in range(G):
            for d in range(1, N_DEV):
                peer = (me + d) % N_DEV
                pltpu.make_async_remote_copy(
                    src_ref=ag_ref.at[pl.ds(g * ROWS_G + peer * SLAB, SLAB), :],
                    dst_ref=ag_ref.at[pl.ds(g * ROWS_G + peer * SLAB, SLAB), :],
                    send_sem=ag_send.at[g, peer],
                    recv_sem=ag_recv.at[g, peer],
                    device_id=(peer,),
                    device_id_type=pl.DeviceIdType.MESH,
                ).wait_recv()

        out_ref[...] = ag_ref[...].astype(jnp.float32)

        for desc in send_descs:
            desc.wait_send()

    out = pl.pallas_call(
        body,
        out_shape=jax.ShapeDtypeStruct((SQ, D_MODEL), jnp.float32),
        in_specs=[pl.BlockSpec(memory_space=pltpu.VMEM)] * 5,
        out_specs=pl.BlockSpec(memory_space=pltpu.VMEM),
        scratch_shapes=[
            pltpu.VMEM((SQ, D_MODEL), jnp.float32),
            pltpu.VMEM((SQ, D_MODEL), jnp.bfloat16),
            pltpu.VMEM((SQ, D_MODEL), jnp.bfloat16),
            pltpu.VMEM((SQ, D_MODEL), jnp.bfloat16),
            pltpu.SemaphoreType.DMA((G, N_DEV)),
            pltpu.SemaphoreType.DMA((G, N_DEV)),
            pltpu.SemaphoreType.DMA((G, N_DEV)),
            pltpu.SemaphoreType.DMA((G, N_DEV)),
        ],
        compiler_params=pltpu.CompilerParams(collective_id=0),
    )(x2, Wq, Wo, Wk, Wv)
    return out.reshape(1, SQ, D_MODEL)
